# Initial kernel scaffold; baseline (speedup 1.0000x reference)
import jax
import jax.numpy as jnp
from jax.experimental import pallas as pl


def kernel(x, Wq, Wk, Wv):
    raise NotImplementedError("write your pallas kernel here")


# speedup vs baseline: 5.3750x; 5.3750x over previous
import jax
import jax.numpy as jnp
from jax.experimental import pallas as pl

_B = 3
_S = 2048
_E = 1024
_HD = 64
_BS = 16
_TILE = 256
_SCALE = _HD ** -0.5


def _paged_attn_kernel(x_ref, x16_ref, wq_ref, wk_ref, wv_ref, out_ref):
    t = pl.program_id(1)
    x_tile = x_ref[0]
    x16 = x16_ref[0]

    dn = (((1,), (1,)), ((), ()))
    q = jax.lax.dot_general(x_tile, wq_ref[:, :], dn,
                            preferred_element_type=jnp.float32)
    k16 = jax.lax.dot_general(x16, wk_ref[:, :], dn,
                              preferred_element_type=jnp.float32)
    v16 = jax.lax.dot_general(x16, wv_ref[:, :], dn,
                              preferred_element_type=jnp.float32)

    s = jax.lax.dot_general(q, k16, dn,
                            preferred_element_type=jnp.float32) * _SCALE

    i = t * _TILE + jax.lax.broadcasted_iota(jnp.int32, (_TILE, _BS), 0)
    m = jax.lax.broadcasted_iota(jnp.int32, (_TILE, _BS), 1)
    valid = m <= i
    cnt = jnp.where(valid, (i - m) // _BS + 1, 0).astype(jnp.float32)

    s = jnp.where(valid, s, -jnp.inf)
    smax = jnp.max(s, axis=1, keepdims=True)
    w = cnt * jnp.exp(s - smax)
    denom = jnp.sum(w, axis=1, keepdims=True)

    o = jax.lax.dot_general(w, v16, (((1,), (0,)), ((), ())),
                            preferred_element_type=jnp.float32)
    out_ref[0] = o / denom


@jax.jit
def kernel(x, Wq, Wk, Wv):
    grid = (_B, _S // _TILE)
    return pl.pallas_call(
        _paged_attn_kernel,
        grid=grid,
        in_specs=[
            pl.BlockSpec((1, _TILE, _E), lambda b, t: (b, t, 0)),
            pl.BlockSpec((1, _BS, _E), lambda b, t: (b, 0, 0)),
            pl.BlockSpec((_HD, _E), lambda b, t: (0, 0)),
            pl.BlockSpec((_HD, _E), lambda b, t: (0, 0)),
            pl.BlockSpec((_HD, _E), lambda b, t: (0, 0)),
        ],
        out_specs=pl.BlockSpec((1, _TILE, _HD), lambda b, t: (b, t, 0)),
        out_shape=jax.ShapeDtypeStruct((_B, _S, _HD), jnp.float32),
    )(x, x[:, :_BS, :], Wq, Wk, Wv)


# speedup vs baseline: 6.9986x; 1.3021x over previous
import jax
import jax.numpy as jnp
from jax.experimental import pallas as pl

_B = 3
_S = 2048
_E = 1024
_HD = 64
_BS = 16
_TILE = 512
_SCALE = _HD ** -0.5


def _paged_attn_kernel(x_ref, x16_ref, wq_ref, wk_ref, wv_ref, out_ref):
    t = pl.program_id(0)
    x_tile = x_ref[:, :]
    x16 = x16_ref[0]

    dn_nt = (((1,), (1,)), ((), ()))
    dn_nn = (((1,), (0,)), ((), ()))
    k16 = jax.lax.dot_general(x16, wk_ref[:, :], dn_nt,
                              preferred_element_type=jnp.float32)
    v16 = jax.lax.dot_general(x16, wv_ref[:, :], dn_nt,
                              preferred_element_type=jnp.float32)
    a = jax.lax.dot_general(k16, wq_ref[:, :], dn_nn,
                            preferred_element_type=jnp.float32)

    s = jax.lax.dot_general(x_tile, a, dn_nt,
                            preferred_element_type=jnp.float32) * _SCALE

    g = t * _TILE + jax.lax.broadcasted_iota(jnp.int32, (_TILE, _BS), 0)
    i = g % _S
    m = jax.lax.broadcasted_iota(jnp.int32, (_TILE, _BS), 1)
    valid = m <= i
    cnt = jnp.where(valid, (i - m) // _BS + 1, 0).astype(jnp.float32)

    s = jnp.where(valid, s, -jnp.inf)
    smax = jnp.max(s, axis=1, keepdims=True)
    w = cnt * jnp.exp(s - smax)
    denom = jnp.sum(w, axis=1, keepdims=True)

    o = jax.lax.dot_general(w, v16, dn_nn,
                            preferred_element_type=jnp.float32)
    out_ref[:, :] = o / denom


@jax.jit
def kernel(x, Wq, Wk, Wv):
    xf = x.reshape(_B * _S, _E)
    tiles_per_req = _S // _TILE
    out = pl.pallas_call(
        _paged_attn_kernel,
        grid=(_B * _S // _TILE,),
        in_specs=[
            pl.BlockSpec((_TILE, _E), lambda t: (t, 0)),
            pl.BlockSpec((1, _BS, _E), lambda t: (t // tiles_per_req, 0, 0)),
            pl.BlockSpec((_HD, _E), lambda t: (0, 0)),
            pl.BlockSpec((_HD, _E), lambda t: (0, 0)),
            pl.BlockSpec((_HD, _E), lambda t: (0, 0)),
        ],
        out_specs=pl.BlockSpec((_TILE, _HD), lambda t: (t, 0)),
        out_shape=jax.ShapeDtypeStruct((_B * _S, _HD), jnp.float32),
    )(xf, x[:, :_BS, :], Wq, Wk, Wv)
    return out.reshape(_B, _S, _HD)


# speedup vs baseline: 9.0894x; 1.2988x over previous
import jax
import jax.numpy as jnp
from jax.experimental import pallas as pl

_B = 3
_S = 2048
_E = 1024
_HD = 64
_BS = 16
_TILE = 1024
_SCALE = _HD ** -0.5


def _paged_attn_kernel(x_ref, x16_ref, wq_ref, wk_ref, wv_ref, out_ref):
    t = pl.program_id(0)
    x_tile = x_ref[:, :]
    x16 = x16_ref[0]

    dn_nt = (((1,), (1,)), ((), ()))
    dn_nn = (((1,), (0,)), ((), ()))
    k16 = jax.lax.dot_general(x16, wk_ref[:, :], dn_nt,
                              preferred_element_type=jnp.float32)
    v16 = jax.lax.dot_general(x16, wv_ref[:, :], dn_nt,
                              preferred_element_type=jnp.float32)
    a = jax.lax.dot_general(k16, wq_ref[:, :], dn_nn,
                            preferred_element_type=jnp.float32)

    s = jax.lax.dot_general(x_tile, a, dn_nt,
                            preferred_element_type=jnp.float32) * _SCALE

    row = jax.lax.broadcasted_iota(jnp.int32, (_TILE, _BS), 0)
    m = jax.lax.broadcasted_iota(jnp.int32, (_TILE, _BS), 1)
    base_d = (t % (_S // _TILE)) * (_TILE // _BS)
    d = base_d + (row >> 4)
    r = row & (_BS - 1)
    cnt = d.astype(jnp.float32) + (m <= r).astype(jnp.float32)

    smax = jnp.max(s, axis=1, keepdims=True)
    w = cnt * jnp.exp(s - smax)
    denom = jnp.sum(w, axis=1, keepdims=True)

    o = jax.lax.dot_general(w, v16, dn_nn,
                            preferred_element_type=jnp.float32)
    out_ref[:, :] = o / denom


@jax.jit
def kernel(x, Wq, Wk, Wv):
    xf = x.reshape(_B * _S, _E)
    tiles_per_req = _S // _TILE
    out = pl.pallas_call(
        _paged_attn_kernel,
        grid=(_B * _S // _TILE,),
        in_specs=[
            pl.BlockSpec((_TILE, _E), lambda t: (t, 0)),
            pl.BlockSpec((1, _BS, _E), lambda t: (t // tiles_per_req, 0, 0)),
            pl.BlockSpec((_HD, _E), lambda t: (0, 0)),
            pl.BlockSpec((_HD, _E), lambda t: (0, 0)),
            pl.BlockSpec((_HD, _E), lambda t: (0, 0)),
        ],
        out_specs=pl.BlockSpec((_TILE, _HD), lambda t: (t, 0)),
        out_shape=jax.ShapeDtypeStruct((_B * _S, _HD), jnp.float32),
    )(xf, x[:, :_BS, :], Wq, Wk, Wv)
    return out.reshape(_B, _S, _HD)


# speedup vs baseline: 9.1512x; 1.0068x over previous
import jax
import jax.numpy as jnp
from jax.experimental import pallas as pl
from jax.experimental.pallas import tpu as pltpu

_B = 3
_S = 2048
_E = 1024
_HD = 64
_BS = 16
_TILE = 1024
_SCALE = _HD ** -0.5


def _paged_attn_kernel(x_ref, x16_ref, wq_ref, wk_ref, wv_ref, out_ref,
                       a_ref, v_ref):
    t = pl.program_id(0)
    tiles_per_req = _S // _TILE

    @pl.when(t % tiles_per_req == 0)
    def _prologue():
        x16 = x16_ref[0]
        dn_nt = (((1,), (1,)), ((), ()))
        dn_nn = (((1,), (0,)), ((), ()))
        k16 = jax.lax.dot_general(x16, wk_ref[:, :], dn_nt,
                                  preferred_element_type=jnp.float32)
        v16 = jax.lax.dot_general(x16, wv_ref[:, :], dn_nt,
                                  preferred_element_type=jnp.float32)
        a_ref[:, :] = jax.lax.dot_general(k16 * _SCALE, wq_ref[:, :], dn_nn,
                                          preferred_element_type=jnp.float32)
        v_ref[:, :] = jnp.concatenate(
            [v16, jnp.ones((_BS, 1), jnp.float32)], axis=1)

    s_t = jax.lax.dot_general(a_ref[:, :], x_ref[:, :],
                              (((1,), (1,)), ((), ())),
                              preferred_element_type=jnp.float32)

    row = jax.lax.broadcasted_iota(jnp.int32, (_BS, _TILE), 1)
    m = jax.lax.broadcasted_iota(jnp.int32, (_BS, _TILE), 0)
    base_d = (t % tiles_per_req) * (_TILE // _BS)
    d = base_d + (row >> 4)
    r = row & (_BS - 1)
    cnt = d.astype(jnp.float32) + (m <= r).astype(jnp.float32)

    smax = jnp.max(s_t, axis=0, keepdims=True)
    w = cnt * jnp.exp(s_t - smax)

    out_aug = jax.lax.dot_general(w, v_ref[:, :],
                                  (((0,), (0,)), ((), ())),
                                  preferred_element_type=jnp.float32)
    out_ref[:, :] = out_aug[:, :_HD] / out_aug[:, _HD:]


@jax.jit
def kernel(x, Wq, Wk, Wv):
    xf = x.reshape(_B * _S, _E)
    tiles_per_req = _S // _TILE
    out = pl.pallas_call(
        _paged_attn_kernel,
        grid=(_B * _S // _TILE,),
        in_specs=[
            pl.BlockSpec((_TILE, _E), lambda t: (t, 0)),
            pl.BlockSpec((1, _BS, _E), lambda t: (t // tiles_per_req, 0, 0)),
            pl.BlockSpec((_HD, _E), lambda t: (0, 0)),
            pl.BlockSpec((_HD, _E), lambda t: (0, 0)),
            pl.BlockSpec((_HD, _E), lambda t: (0, 0)),
        ],
        out_specs=pl.BlockSpec((_TILE, _HD), lambda t: (t, 0)),
        out_shape=jax.ShapeDtypeStruct((_B * _S, _HD), jnp.float32),
        scratch_shapes=[
            pltpu.VMEM((_BS, _E), jnp.float32),
            pltpu.VMEM((_BS, _HD + 1), jnp.float32),
        ],
    )(xf, x[:, :_BS, :], Wq, Wk, Wv)
    return out.reshape(_B, _S, _HD)


# speedup vs baseline: 9.5483x; 1.0434x over previous
import jax
import jax.numpy as jnp
from jax.experimental import pallas as pl
from jax.experimental.pallas import tpu as pltpu

_B = 3
_S = 2048
_E = 1024
_HD = 64
_BS = 16
_TILE = 2048
_SCALE = _HD ** -0.5


def _paged_attn_kernel(x_ref, x16_ref, wq_ref, wk_ref, wv_ref, out_ref,
                       a_ref, v_ref):
    t = pl.program_id(0)
    tiles_per_req = _S // _TILE

    @pl.when(t % tiles_per_req == 0)
    def _prologue():
        x16 = x16_ref[0]
        dn_nt = (((1,), (1,)), ((), ()))
        dn_nn = (((1,), (0,)), ((), ()))
        k16 = jax.lax.dot_general(x16, wk_ref[:, :], dn_nt,
                                  preferred_element_type=jnp.float32)
        v16 = jax.lax.dot_general(x16, wv_ref[:, :], dn_nt,
                                  preferred_element_type=jnp.float32)
        a_ref[:, :] = jax.lax.dot_general(k16 * _SCALE, wq_ref[:, :], dn_nn,
                                          preferred_element_type=jnp.float32)
        v_ref[:, :] = jnp.concatenate(
            [v16, jnp.ones((_BS, 1), jnp.float32)], axis=1)

    s_t = jax.lax.dot_general(a_ref[:, :], x_ref[:, :],
                              (((1,), (1,)), ((), ())),
                              preferred_element_type=jnp.float32)

    row = jax.lax.broadcasted_iota(jnp.int32, (_BS, _TILE), 1)
    m = jax.lax.broadcasted_iota(jnp.int32, (_BS, _TILE), 0)
    base_d = (t % tiles_per_req) * (_TILE // _BS)
    d = base_d + (row >> 4)
    r = row & (_BS - 1)
    cnt = d.astype(jnp.float32) + (m <= r).astype(jnp.float32)

    smax = jnp.max(s_t, axis=0, keepdims=True)
    w = cnt * jnp.exp(s_t - smax)

    out_aug = jax.lax.dot_general(w, v_ref[:, :],
                                  (((0,), (0,)), ((), ())),
                                  preferred_element_type=jnp.float32)
    out_ref[:, :] = out_aug[:, :_HD] / out_aug[:, _HD:]


@jax.jit
def kernel(x, Wq, Wk, Wv):
    xf = x.reshape(_B * _S, _E)
    tiles_per_req = _S // _TILE
    out = pl.pallas_call(
        _paged_attn_kernel,
        grid=(_B * _S // _TILE,),
        in_specs=[
            pl.BlockSpec((_TILE, _E), lambda t: (t, 0)),
            pl.BlockSpec((1, _BS, _E), lambda t: (t // tiles_per_req, 0, 0)),
            pl.BlockSpec((_HD, _E), lambda t: (0, 0)),
            pl.BlockSpec((_HD, _E), lambda t: (0, 0)),
            pl.BlockSpec((_HD, _E), lambda t: (0, 0)),
        ],
        out_specs=pl.BlockSpec((_TILE, _HD), lambda t: (t, 0)),
        out_shape=jax.ShapeDtypeStruct((_B * _S, _HD), jnp.float32),
        scratch_shapes=[
            pltpu.VMEM((_BS, _E), jnp.float32),
            pltpu.VMEM((_BS, _HD + 1), jnp.float32),
        ],
    )(xf, x[:, :_BS, :], Wq, Wk, Wv)
    return out.reshape(_B, _S, _HD)
